# Initial kernel scaffold; baseline (speedup 1.0000x reference)
import jax
import jax.numpy as jnp
from jax.experimental import pallas as pl


def kernel(x, adj, W1, W2, W3):
    raise NotImplementedError("write your pallas kernel here")


# speedup vs baseline: 1.3304x; 1.3304x over previous
import jax
import jax.numpy as jnp
from jax.experimental import pallas as pl
from jax.experimental.pallas import tpu as pltpu

_BM = 400


def _s1_body(x_ref, w_ref, o_ref):
    o_ref[...] = jnp.dot(x_ref[...], w_ref[...],
                         preferred_element_type=jnp.float32)


def _pass1_body(adj_ref, s1_ref, wc_ref, o_ref):
    h = jnp.dot(adj_ref[...], s1_ref[...],
                preferred_element_type=jnp.float32)
    h = jnp.maximum(h, 0.0)
    o_ref[...] = jnp.dot(h, wc_ref[...],
                         preferred_element_type=jnp.float32)


def _pass2_body(adj_ref, s2_ref, o_ref):
    o = jnp.dot(adj_ref[...], s2_ref[...],
                preferred_element_type=jnp.float32)
    o_ref[...] = jnp.maximum(o, 0.0)


def kernel(x, adj, W1, W2, W3):
    n, _ = x.shape
    h1 = W1.shape[1]
    h2 = W2.shape[1]
    wc = jnp.concatenate([W2, W3], axis=1)
    bm = _BM
    grid = (n // bm,)

    s1 = pl.pallas_call(
        _s1_body,
        out_shape=jax.ShapeDtypeStruct((n, h1), jnp.float32),
    )(x, W1)

    s2 = pl.pallas_call(
        _pass1_body,
        grid=grid,
        in_specs=[
            pl.BlockSpec((bm, n), lambda m: (m, 0)),
            pl.BlockSpec((n, h1), lambda m: (0, 0)),
            pl.BlockSpec((h1, 2 * h2), lambda m: (0, 0)),
        ],
        out_specs=pl.BlockSpec((bm, 2 * h2), lambda m: (m, 0)),
        out_shape=jax.ShapeDtypeStruct((n, 2 * h2), jnp.float32),
        compiler_params=pltpu.CompilerParams(
            dimension_semantics=("parallel",)),
    )(adj, s1, wc)

    out2 = pl.pallas_call(
        _pass2_body,
        grid=grid,
        in_specs=[
            pl.BlockSpec((bm, n), lambda m: (m, 0)),
            pl.BlockSpec((n, 2 * h2), lambda m: (0, 0)),
        ],
        out_specs=pl.BlockSpec((bm, 2 * h2), lambda m: (m, 0)),
        out_shape=jax.ShapeDtypeStruct((n, 2 * h2), jnp.float32),
        compiler_params=pltpu.CompilerParams(
            dimension_semantics=("parallel",)),
    )(adj, s2)

    mu = out2[:, :h2]
    logvar = out2[:, h2:]
    return (mu, mu, logvar)
